# baseline (speedup 1.0000x reference)
import jax
import jax.numpy as jnp
from jax.experimental import pallas as pl
from jax.experimental.pallas import tpu as pltpu

_BR = 80
_Q = 3


def _attention_combine(ps, a):
    cols = [jnp.dot(p, a[:, s:s + 1], preferred_element_type=jnp.float32)
            for s, p in enumerate(ps)]
    scores = jnp.concatenate(cols, axis=1)
    scores = jnp.where(scores >= 0, scores, 0.2 * scores)
    m = jnp.max(scores, axis=1, keepdims=True)
    e = jnp.exp(scores - m)
    alpha = e / jnp.sum(e, axis=1, keepdims=True)
    out = ps[0] * alpha[:, 0:1]
    for s in range(1, 4):
        out = out + ps[s] * alpha[:, s:s + 1]
    return jnp.maximum(out, 0.0)


def _proj_body(x_ref, W_ref, o_ref):
    o_ref[...] = jnp.dot(x_ref[...], W_ref[...],
                         preferred_element_type=jnp.float32).astype(
                             jnp.bfloat16)


def _body(hp1_ref, A_ref, s1_ref, s2_ref, s3_ref, a1_ref,
          W2_ref, a2_ref, out_ref, buf_ref, hp2_ref, sems):
    t = pl.program_id(0)
    T = pl.num_programs(0)
    R = T // 2
    hbms = (A_ref, s1_ref, s2_ref, s3_ref)

    def copies(u):
        slot = u % _Q
        iu = u % R
        return [pltpu.make_async_copy(
            m.at[pl.ds(iu * _BR, _BR), :],
            buf_ref.at[slot, pl.ds(s * _BR, _BR), :],
            sems.at[slot, s]) for s, m in enumerate(hbms)]

    @pl.when(t == 0)
    def _prologue():
        for u in range(_Q):
            for c in copies(u):
                c.start()

    for c in copies(t):
        c.wait()
    slot = t % _Q
    slab = buf_ref[slot]
    i = t % R

    def prop4(hp):
        pcat = jax.lax.dot_general(slab, hp, (((1,), (0,)), ((), ())),
                                   preferred_element_type=jnp.float32)
        return [pcat[s * _BR:(s + 1) * _BR] for s in range(4)]

    @pl.when(t < R)
    def _layer1():
        h1_blk = _attention_combine(prop4(hp1_ref[...]), a1_ref[...])
        hp2_ref[pl.ds(i * _BR, _BR), :] = jnp.dot(
            h1_blk, W2_ref[...],
            preferred_element_type=jnp.float32).astype(jnp.bfloat16)

    @pl.when(t >= R)
    def _layer2():
        out = _attention_combine(prop4(hp2_ref[...]), a2_ref[...])
        mx = jnp.max(out, axis=1, keepdims=True)
        shifted = out - mx
        lse = jnp.log(jnp.sum(jnp.exp(shifted), axis=1, keepdims=True))
        out_ref[...] = shifted - lse

    @pl.when(t + _Q < T)
    def _prefetch():
        for c in copies(t + _Q):
            c.start()


def kernel(x, A_tilde, s1_sct, s2_sct, s3_sct, W1, a1, W2, a2):
    N, NFEAT = x.shape
    HID = W1.shape[1]
    NCLASS = W2.shape[1]
    R = N // _BR
    hbm_spec = pl.BlockSpec(memory_space=pltpu.HBM)

    hp1 = pl.pallas_call(
        _proj_body,
        out_shape=jax.ShapeDtypeStruct((N, HID), jnp.bfloat16),
    )(x, W1)

    def full(shape):
        return pl.BlockSpec(shape, lambda t: (0, 0))

    out = pl.pallas_call(
        _body,
        grid=(2 * R,),
        in_specs=[full((N, HID)), hbm_spec, hbm_spec, hbm_spec, hbm_spec,
                  full((HID, 4)),
                  full((HID, NCLASS)), full((NCLASS, 4))],
        out_specs=pl.BlockSpec(
            (_BR, NCLASS), lambda t: (jnp.where(t < N // _BR,
                                                N // _BR, t - N // _BR), 0)),
        out_shape=jax.ShapeDtypeStruct((N + _BR, NCLASS), jnp.float32),
        scratch_shapes=[pltpu.VMEM((_Q, 4 * _BR, N), jnp.float32),
                        pltpu.VMEM((N, NCLASS), jnp.bfloat16),
                        pltpu.SemaphoreType.DMA((_Q, 4))],
        compiler_params=pltpu.CompilerParams(
            dimension_semantics=("arbitrary",)),
    )(hp1, A_tilde, s1_sct, s2_sct, s3_sct, a1, W2, a2)
    return out[:N]
